# Initial kernel scaffold; baseline (speedup 1.0000x reference)
import functools

import jax
import jax.numpy as jnp
from jax import lax
from jax.experimental import pallas as pl
from jax.experimental.pallas import tpu as pltpu
from jax.experimental.pallas import tpu_sc as plsc

N, E, D, H, C = 10000, 320000, 128, 128, 64
NC, NS = 2, 16
NW = NC * NS
B = 128
K = -(-E // (NW * B))
E_PAD = NW * K * B
N_PAD = 10016
ZROWS = N_PAD // NS
OROWS = N // NS

_sc_mesh = plsc.VectorSubcoreMesh(
    core_axis_name="c", subcore_axis_name="s", num_cores=NC, num_subcores=NS)


@functools.partial(
    pl.kernel,
    out_type=jax.ShapeDtypeStruct((NC, N, D), jnp.float32),
    mesh=_sc_mesh,
    scratch_types=[
        pltpu.VMEM((K, B), jnp.int32),
        pltpu.VMEM((K, B), jnp.int32),
        pltpu.VMEM((B, D), jnp.float32),
        pltpu.VMEM_SHARED((N_PAD, D), jnp.float32),
        pltpu.SemaphoreType.DMA,
    ],
)
def _sc_agg(x_hbm, src_hbm, dst_hbm, zeros_hbm, out_hbm,
            src_v, dst_v, rows_v, acc_sh, sem):
    c = lax.axis_index("c")
    s = lax.axis_index("s")
    wid = s * NC + c
    pltpu.sync_copy(zeros_hbm.at[pl.ds(s * ZROWS, ZROWS)],
                    acc_sh.at[pl.ds(s * ZROWS, ZROWS)])
    pltpu.sync_copy(src_hbm.at[wid], src_v)
    pltpu.sync_copy(dst_hbm.at[wid], dst_v)
    plsc.subcore_barrier()

    def body(j, carry):
        pltpu.async_copy(x_hbm.at[src_v.at[j]], rows_v, sem).wait()
        pltpu.sync_copy(rows_v, acc_sh.at[dst_v.at[j]], add=True)
        return carry

    lax.fori_loop(0, K, body, 0)
    plsc.subcore_barrier()
    pltpu.sync_copy(acc_sh.at[pl.ds(s * OROWS, OROWS)],
                    out_hbm.at[c, pl.ds(s * OROWS, OROWS)])


def _mlp_body(p_ref, x_ref, w1_ref, b1_ref, g_ref, be_ref, w2_ref, b2_ref,
              o_ref):
    h = p_ref[0] + p_ref[1] + x_ref[...]
    h = jnp.dot(h, w1_ref[...], preferred_element_type=jnp.float32) + b1_ref[...]
    m = jnp.mean(h, axis=0, keepdims=True)
    cen = h - m
    v = jnp.mean(cen * cen, axis=0, keepdims=True)
    h = g_ref[...] * cen * lax.rsqrt(v + 1e-5) + be_ref[...]
    h = jnp.maximum(h, 0.0)
    h = jnp.dot(h, w2_ref[...], preferred_element_type=jnp.float32) + b2_ref[...]
    o_ref[...] = jnp.maximum(h, 0.0)


_mlp = pl.pallas_call(
    _mlp_body,
    out_shape=jax.ShapeDtypeStruct((N, H), jnp.float32),
)


def _mlp_head_body(p_ref, x_ref, w1_ref, b1_ref, g_ref, be_ref, w2_ref,
                   b2_ref, l1w_ref, l1b_ref, l2w_ref, l2b_ref, o_ref):
    h = p_ref[0] + p_ref[1] + x_ref[...]
    h = jnp.dot(h, w1_ref[...], preferred_element_type=jnp.float32) + b1_ref[...]
    m = jnp.mean(h, axis=0, keepdims=True)
    cen = h - m
    v = jnp.mean(cen * cen, axis=0, keepdims=True)
    h = g_ref[...] * cen * lax.rsqrt(v + 1e-5) + be_ref[...]
    h = jnp.maximum(h, 0.0)
    h = jnp.dot(h, w2_ref[...], preferred_element_type=jnp.float32) + b2_ref[...]
    h = jnp.maximum(h, 0.0)
    h = jnp.dot(h, l1w_ref[...], preferred_element_type=jnp.float32) + l1b_ref[...]
    h = jnp.maximum(h, 0.0)
    h = jnp.dot(h, l2w_ref[...], preferred_element_type=jnp.float32) + l2b_ref[...]
    o_ref[...] = jax.nn.sigmoid(h)


_mlp_head = pl.pallas_call(
    _mlp_head_body,
    out_shape=jax.ShapeDtypeStruct((N, C), jnp.float32),
)


@jax.jit
def kernel(x, edge_index, w1_0, b1_0, g_0, be_0, w2_0, b2_0,
           w1_1, b1_1, g_1, be_1, w2_1, b2_1,
           w1_2, b1_2, g_2, be_2, w2_2, b2_2,
           lin1_w, lin1_b, lin2_w, lin2_b):
    x = x.astype(jnp.float32)
    pad = E_PAD - E
    src3 = jnp.concatenate(
        [edge_index[0], jnp.zeros((pad,), jnp.int32)]).reshape(NW, K, B)
    dst3 = jnp.concatenate(
        [edge_index[1], jnp.full((pad,), N, jnp.int32)]).reshape(NW, K, B)
    zeros = jnp.zeros((N_PAD, D), jnp.float32)

    layers = [
        (w1_0, b1_0, g_0, be_0, w2_0, b2_0),
        (w1_1, b1_1, g_1, be_1, w2_1, b2_1),
        (w1_2, b1_2, g_2, be_2, w2_2, b2_2),
    ]
    h = x
    for i, (w1, b1, g, be, w2, b2) in enumerate(layers):
        p = _sc_agg(h, src3, dst3, zeros)
        if i < 2:
            h = _mlp(p, h, w1, b1.reshape(1, H), g.reshape(1, H),
                     be.reshape(1, H), w2, b2.reshape(1, H))
        else:
            h = _mlp_head(p, h, w1, b1.reshape(1, H), g.reshape(1, H),
                          be.reshape(1, H), w2, b2.reshape(1, H),
                          lin1_w, lin1_b.reshape(1, H),
                          lin2_w, lin2_b.reshape(1, C))
    return h


# speedup vs baseline: 2.8490x; 2.8490x over previous
import functools

import jax
import jax.numpy as jnp
from jax import lax
from jax.experimental import pallas as pl
from jax.experimental.pallas import tpu as pltpu
from jax.experimental.pallas import tpu_sc as plsc

N, E, D, H, C = 10000, 320000, 128, 128, 64
NC, NS = 2, 16
NW = NC * NS
B = 128
K = 80
E_PAD = NW * K * B
N_PAD = 10112
ZROWS = N_PAD // NS
OROWS = N_PAD // NS

@functools.lru_cache(maxsize=None)
def _get_sc_agg():
    mesh = plsc.VectorSubcoreMesh(
        core_axis_name="c", subcore_axis_name="s",
        num_cores=NC, num_subcores=NS)

    @functools.partial(
        pl.kernel,
        out_type=jax.ShapeDtypeStruct((NC, N_PAD, D), jnp.float32),
        mesh=mesh,
        scratch_types=[
            pltpu.VMEM((K, B), jnp.int32),
            pltpu.VMEM((K, B), jnp.int32),
            pltpu.VMEM((B, D), jnp.float32),
            pltpu.VMEM_SHARED((N_PAD, D), jnp.float32),
            pltpu.SemaphoreType.DMA,
        ],
    )
    def _sc_agg(x_hbm, src_hbm, dst_hbm, zeros_hbm, out_hbm,
                src_v, dst_v, rows_v, acc_sh, sem):
        c = lax.axis_index("c")
        s = lax.axis_index("s")
        wid = s * NC + c
        pltpu.sync_copy(zeros_hbm.at[pl.ds(s * ZROWS, ZROWS)],
                        acc_sh.at[pl.ds(s * ZROWS, ZROWS)])
        pltpu.sync_copy(src_hbm.at[wid], src_v)
        pltpu.sync_copy(dst_hbm.at[wid], dst_v)
        plsc.subcore_barrier()

        def body(j, carry):
            pltpu.async_copy(x_hbm.at[src_v.at[j]], rows_v, sem).wait()
            pltpu.sync_copy(rows_v, acc_sh.at[dst_v.at[j]], add=True)
            return carry

        lax.fori_loop(0, K, body, 0)
        plsc.subcore_barrier()
        pltpu.sync_copy(acc_sh.at[pl.ds(s * OROWS, OROWS)],
                        out_hbm.at[c, pl.ds(s * OROWS, OROWS)])

    return _sc_agg


def _mlp_body(p_ref, x_ref, w1_ref, b1_ref, g_ref, be_ref, w2_ref, b2_ref,
              o_ref):
    h = p_ref[0, :N] + p_ref[1, :N] + x_ref[...]
    h = jnp.dot(h, w1_ref[...], preferred_element_type=jnp.float32) + b1_ref[...]
    m = jnp.mean(h, axis=0, keepdims=True)
    cen = h - m
    v = jnp.mean(cen * cen, axis=0, keepdims=True)
    h = g_ref[...] * cen * lax.rsqrt(v + 1e-5) + be_ref[...]
    h = jnp.maximum(h, 0.0)
    h = jnp.dot(h, w2_ref[...], preferred_element_type=jnp.float32) + b2_ref[...]
    o_ref[...] = jnp.maximum(h, 0.0)


_mlp = pl.pallas_call(
    _mlp_body,
    out_shape=jax.ShapeDtypeStruct((N, H), jnp.float32),
)


def _mlp_head_body(p_ref, x_ref, w1_ref, b1_ref, g_ref, be_ref, w2_ref,
                   b2_ref, l1w_ref, l1b_ref, l2w_ref, l2b_ref, o_ref):
    h = p_ref[0, :N] + p_ref[1, :N] + x_ref[...]
    h = jnp.dot(h, w1_ref[...], preferred_element_type=jnp.float32) + b1_ref[...]
    m = jnp.mean(h, axis=0, keepdims=True)
    cen = h - m
    v = jnp.mean(cen * cen, axis=0, keepdims=True)
    h = g_ref[...] * cen * lax.rsqrt(v + 1e-5) + be_ref[...]
    h = jnp.maximum(h, 0.0)
    h = jnp.dot(h, w2_ref[...], preferred_element_type=jnp.float32) + b2_ref[...]
    h = jnp.maximum(h, 0.0)
    h = jnp.dot(h, l1w_ref[...], preferred_element_type=jnp.float32) + l1b_ref[...]
    h = jnp.maximum(h, 0.0)
    h = jnp.dot(h, l2w_ref[...], preferred_element_type=jnp.float32) + l2b_ref[...]
    o_ref[...] = jax.nn.sigmoid(h)


_mlp_head = pl.pallas_call(
    _mlp_head_body,
    out_shape=jax.ShapeDtypeStruct((N, C), jnp.float32),
)


@jax.jit
def kernel(x, edge_index, w1_0, b1_0, g_0, be_0, w2_0, b2_0,
           w1_1, b1_1, g_1, be_1, w2_1, b2_1,
           w1_2, b1_2, g_2, be_2, w2_2, b2_2,
           lin1_w, lin1_b, lin2_w, lin2_b):
    x = x.astype(jnp.float32)
    pad = E_PAD - E
    src3 = jnp.concatenate(
        [edge_index[0], jnp.zeros((pad,), jnp.int32)]).reshape(NW, K, B)
    dst3 = jnp.concatenate(
        [edge_index[1], jnp.full((pad,), N, jnp.int32)]).reshape(NW, K, B)
    zeros = jnp.zeros((N_PAD, D), jnp.float32)

    layers = [
        (w1_0, b1_0, g_0, be_0, w2_0, b2_0),
        (w1_1, b1_1, g_1, be_1, w2_1, b2_1),
        (w1_2, b1_2, g_2, be_2, w2_2, b2_2),
    ]
    sc_agg = _get_sc_agg()
    h = x
    for i, (w1, b1, g, be, w2, b2) in enumerate(layers):
        p = sc_agg(h, src3, dst3, zeros)
        if i < 2:
            h = _mlp(p, h, w1, b1.reshape(1, H), g.reshape(1, H),
                     be.reshape(1, H), w2, b2.reshape(1, H))
        else:
            h = _mlp_head(p, h, w1, b1.reshape(1, H), g.reshape(1, H),
                          be.reshape(1, H), w2, b2.reshape(1, H),
                          lin1_w, lin1_b.reshape(1, H),
                          lin2_w, lin2_b.reshape(1, C))
    return h
